# baseline (speedup 1.0000x reference)
import functools

import jax
import jax.numpy as jnp
from jax import lax
from jax.experimental import pallas as pl
from jax.experimental.pallas import tpu as pltpu
from jax.experimental.pallas import tpu_sc as plsc

E = 800000
D = 64
NUM_CORES = 2
NUM_SUBCORES = 16
NW = NUM_CORES * NUM_SUBCORES
CB = 320
T = E // CB
Q, R = divmod(T, NW)
MAXT = Q + 1
GROUPS = MAXT * CB // 16
NREP = 256
NSPLIT = 8


@jax.jit
def _sc_embed(idx, tab128):
    mesh = plsc.VectorSubcoreMesh(core_axis_name="c", subcore_axis_name="s")

    @functools.partial(
        pl.kernel,
        mesh=mesh,
        out_type=jax.ShapeDtypeStruct((E, 2 * D), jnp.float32),
        scratch_types=[
            pltpu.VMEM((MAXT * CB,), jnp.int32),
            pltpu.VMEM((2 * CB, 2 * D), jnp.float32),
            [pltpu.SemaphoreType.DMA] * (2 * NSPLIT),
            pltpu.SemaphoreType.DMA,
            pltpu.SemaphoreType.DMA,
        ],
        compiler_params=pltpu.CompilerParams(needs_layout_passes=False),
    )
    def k(idx_hbm, tab_hbm, out_hbm, idx_v, rows_v, gsems, w0, w1):
        wid = lax.axis_index("s") * NUM_CORES + lax.axis_index("c")
        start = jnp.minimum(wid * Q + jnp.minimum(wid, R), T - MAXT)

        pltpu.sync_copy(idx_hbm.at[pl.ds(start * CB, MAXT * CB)], idx_v)

        iota = lax.iota(jnp.int32, 16)

        def rewrite_body(g, carry):
            v = idx_v[pl.ds(g * 16, 16)]
            rep = jnp.bitwise_and((wid * GROUPS + g) * 16 + iota, NREP - 1)
            idx_v[pl.ds(g * 16, 16)] = jnp.bitwise_and(v, 3) + rep * 4
            return carry

        lax.fori_loop(0, GROUPS, rewrite_body, 0)

        wsem = (w0, w1)
        SP = CB // NSPLIT

        def gather(ci, b):
            descs = [
                pltpu.async_copy(
                    tab_hbm.at[idx_v.at[pl.ds(ci * CB + q * SP, SP)]],
                    rows_v.at[pl.ds(b * CB + q * SP, SP)],
                    gsems[b * NSPLIT + q],
                )
                for q in range(NSPLIT)
            ]

            class _Multi:
                def wait(self):
                    for d in descs:
                        d.wait()

            return _Multi()

        def write(ci, b):
            return pltpu.async_copy(
                rows_v.at[pl.ds(b * CB, CB)],
                out_hbm.at[pl.ds((start + ci) * CB, CB)],
                wsem[b],
            )

        g_desc = [gather(0, 0), None]
        w_desc = [None, None]
        for ci in range(MAXT):
            b = ci & 1
            g_desc[b].wait()
            if ci + 1 < MAXT:
                ob = 1 - b
                if w_desc[ob] is not None:
                    w_desc[ob].wait()
                g_desc[ob] = gather(ci + 1, ob)
            w_desc[b] = write(ci, b)
        w_desc[(MAXT - 1) & 1].wait()
        w_desc[(MAXT - 2) & 1].wait()

    return k(idx, tab128)


def kernel(edge_type, table):
    idx = edge_type.astype(jnp.int32)
    tab128 = jnp.pad(table, ((0, 0), (0, D)))
    tab128 = jnp.tile(tab128, (NREP, 1))
    out3 = _sc_embed(idx, tab128)
    return out3[:, :D]


# speedup vs baseline: 1.1745x; 1.1745x over previous
import functools

import jax
import jax.numpy as jnp
from jax import lax
from jax.experimental import pallas as pl
from jax.experimental.pallas import tpu as pltpu
from jax.experimental.pallas import tpu_sc as plsc

E = 800000
D = 64
NUM_CORES = 2
NUM_SUBCORES = 16
NW = NUM_CORES * NUM_SUBCORES
CB = 320
T = E // CB
Q, R = divmod(T, NW)
MAXT = Q + 1
GROUPS = MAXT * CB // 16
NREP = 512
NSPLIT = 4


@jax.jit
def _sc_embed(idx, tab128):
    mesh = plsc.VectorSubcoreMesh(core_axis_name="c", subcore_axis_name="s")

    @functools.partial(
        pl.kernel,
        mesh=mesh,
        out_type=jax.ShapeDtypeStruct((E, 2 * D), jnp.float32),
        scratch_types=[
            pltpu.VMEM((MAXT * CB,), jnp.int32),
            pltpu.VMEM((2 * CB, 2 * D), jnp.float32),
            [pltpu.SemaphoreType.DMA] * (2 * NSPLIT),
            pltpu.SemaphoreType.DMA,
            pltpu.SemaphoreType.DMA,
        ],
        compiler_params=pltpu.CompilerParams(needs_layout_passes=False),
    )
    def k(idx_hbm, tab_hbm, out_hbm, idx_v, rows_v, gsems, w0, w1):
        wid = lax.axis_index("s") * NUM_CORES + lax.axis_index("c")
        start = jnp.minimum(wid * Q + jnp.minimum(wid, R), T - MAXT)

        pltpu.sync_copy(idx_hbm.at[pl.ds(start * CB, MAXT * CB)], idx_v)

        iota = lax.iota(jnp.int32, 16)

        def rewrite_body(g, carry):
            v = idx_v[pl.ds(g * 16, 16)]
            rep = jnp.bitwise_and((wid * GROUPS + g) * 16 + iota, NREP - 1)
            idx_v[pl.ds(g * 16, 16)] = jnp.bitwise_and(v, 3) + rep * 4
            return carry

        lax.fori_loop(0, GROUPS, rewrite_body, 0)

        wsem = (w0, w1)
        SP = CB // NSPLIT

        def gather(ci, b):
            descs = [
                pltpu.async_copy(
                    tab_hbm.at[idx_v.at[pl.ds(ci * CB + q * SP, SP)]],
                    rows_v.at[pl.ds(b * CB + q * SP, SP)],
                    gsems[b * NSPLIT + q],
                )
                for q in range(NSPLIT)
            ]

            class _Multi:
                def wait(self):
                    for d in descs:
                        d.wait()

            return _Multi()

        def write(ci, b):
            return pltpu.async_copy(
                rows_v.at[pl.ds(b * CB, CB)],
                out_hbm.at[pl.ds((start + ci) * CB, CB)],
                wsem[b],
            )

        g_desc = [gather(0, 0), None]
        w_desc = [None, None]
        for ci in range(MAXT):
            b = ci & 1
            g_desc[b].wait()
            if ci + 1 < MAXT:
                ob = 1 - b
                if w_desc[ob] is not None:
                    w_desc[ob].wait()
                g_desc[ob] = gather(ci + 1, ob)
            w_desc[b] = write(ci, b)
        w_desc[(MAXT - 1) & 1].wait()
        w_desc[(MAXT - 2) & 1].wait()

    return k(idx, tab128)


def kernel(edge_type, table):
    idx = edge_type.astype(jnp.int32)
    tab128 = jnp.pad(table, ((0, 0), (0, D)))
    tab128 = jnp.tile(tab128, (NREP, 1))
    out3 = _sc_embed(idx, tab128)
    return out3[:, :D]


# speedup vs baseline: 1.2732x; 1.0840x over previous
import functools

import jax
import jax.numpy as jnp
from jax import lax
from jax.experimental import pallas as pl
from jax.experimental.pallas import tpu as pltpu
from jax.experimental.pallas import tpu_sc as plsc

E = 800000
D = 64
NUM_CORES = 2
NUM_SUBCORES = 16
NW = NUM_CORES * NUM_SUBCORES
CB = 320
T = E // CB
Q, R = divmod(T, NW)
MAXT = Q + 1
GROUPS = MAXT * CB // 16
NREP = 1024
NSPLIT = 4


@jax.jit
def _sc_embed(idx, tab128):
    mesh = plsc.VectorSubcoreMesh(core_axis_name="c", subcore_axis_name="s")

    @functools.partial(
        pl.kernel,
        mesh=mesh,
        out_type=jax.ShapeDtypeStruct((E, 2 * D), jnp.float32),
        scratch_types=[
            pltpu.VMEM((MAXT * CB,), jnp.int32),
            pltpu.VMEM((2 * CB, 2 * D), jnp.float32),
            [pltpu.SemaphoreType.DMA] * (2 * NSPLIT),
            pltpu.SemaphoreType.DMA,
            pltpu.SemaphoreType.DMA,
        ],
        compiler_params=pltpu.CompilerParams(needs_layout_passes=False),
    )
    def k(idx_hbm, tab_hbm, out_hbm, idx_v, rows_v, gsems, w0, w1):
        wid = lax.axis_index("s") * NUM_CORES + lax.axis_index("c")
        start = jnp.minimum(wid * Q + jnp.minimum(wid, R), T - MAXT)

        pltpu.sync_copy(idx_hbm.at[pl.ds(start * CB, MAXT * CB)], idx_v)

        iota = lax.iota(jnp.int32, 16)

        def rewrite_body(g, carry):
            v = idx_v[pl.ds(g * 16, 16)]
            rep = jnp.bitwise_and((wid * GROUPS + g) * 16 + iota, NREP - 1)
            idx_v[pl.ds(g * 16, 16)] = jnp.bitwise_and(v, 3) + rep * 4
            return carry

        lax.fori_loop(0, GROUPS, rewrite_body, 0)

        wsem = (w0, w1)
        SP = CB // NSPLIT

        def gather(ci, b):
            descs = [
                pltpu.async_copy(
                    tab_hbm.at[idx_v.at[pl.ds(ci * CB + q * SP, SP)]],
                    rows_v.at[pl.ds(b * CB + q * SP, SP)],
                    gsems[b * NSPLIT + q],
                )
                for q in range(NSPLIT)
            ]

            class _Multi:
                def wait(self):
                    for d in descs:
                        d.wait()

            return _Multi()

        def write(ci, b):
            return pltpu.async_copy(
                rows_v.at[pl.ds(b * CB, CB)],
                out_hbm.at[pl.ds((start + ci) * CB, CB)],
                wsem[b],
            )

        g_desc = [gather(0, 0), None]
        w_desc = [None, None]
        for ci in range(MAXT):
            b = ci & 1
            g_desc[b].wait()
            if ci + 1 < MAXT:
                ob = 1 - b
                if w_desc[ob] is not None:
                    w_desc[ob].wait()
                g_desc[ob] = gather(ci + 1, ob)
            w_desc[b] = write(ci, b)
        w_desc[(MAXT - 1) & 1].wait()
        w_desc[(MAXT - 2) & 1].wait()

    return k(idx, tab128)


def kernel(edge_type, table):
    idx = edge_type.astype(jnp.int32)
    tab128 = jnp.pad(table, ((0, 0), (0, D)))
    tab128 = jnp.tile(tab128, (NREP, 1))
    out3 = _sc_embed(idx, tab128)
    return out3[:, :D]


# speedup vs baseline: 1.3212x; 1.0377x over previous
import functools

import jax
import jax.numpy as jnp
from jax import lax
from jax.experimental import pallas as pl
from jax.experimental.pallas import tpu as pltpu
from jax.experimental.pallas import tpu_sc as plsc

E = 800000
D = 64
NUM_CORES = 2
NUM_SUBCORES = 16
NW = NUM_CORES * NUM_SUBCORES
CB = 320
T = E // CB
Q, R = divmod(T, NW)
MAXT = Q + 1
GROUPS = MAXT * CB // 16
NREP = 2048
NSPLIT = 4


@jax.jit
def _sc_embed(idx, tab128):
    mesh = plsc.VectorSubcoreMesh(core_axis_name="c", subcore_axis_name="s")

    @functools.partial(
        pl.kernel,
        mesh=mesh,
        out_type=jax.ShapeDtypeStruct((E, 2 * D), jnp.float32),
        scratch_types=[
            pltpu.VMEM((MAXT * CB,), jnp.int32),
            pltpu.VMEM((2 * CB, 2 * D), jnp.float32),
            [pltpu.SemaphoreType.DMA] * (2 * NSPLIT),
            pltpu.SemaphoreType.DMA,
            pltpu.SemaphoreType.DMA,
        ],
        compiler_params=pltpu.CompilerParams(needs_layout_passes=False),
    )
    def k(idx_hbm, tab_hbm, out_hbm, idx_v, rows_v, gsems, w0, w1):
        wid = lax.axis_index("s") * NUM_CORES + lax.axis_index("c")
        start = jnp.minimum(wid * Q + jnp.minimum(wid, R), T - MAXT)

        pltpu.sync_copy(idx_hbm.at[pl.ds(start * CB, MAXT * CB)], idx_v)

        iota = lax.iota(jnp.int32, 16)

        def rewrite_body(g, carry):
            v = idx_v[pl.ds(g * 16, 16)]
            rep = jnp.bitwise_and((wid * GROUPS + g) * 16 + iota, NREP - 1)
            idx_v[pl.ds(g * 16, 16)] = jnp.bitwise_and(v, 3) + rep * 4
            return carry

        lax.fori_loop(0, GROUPS, rewrite_body, 0)

        wsem = (w0, w1)
        SP = CB // NSPLIT

        def gather(ci, b):
            descs = [
                pltpu.async_copy(
                    tab_hbm.at[idx_v.at[pl.ds(ci * CB + q * SP, SP)]],
                    rows_v.at[pl.ds(b * CB + q * SP, SP)],
                    gsems[b * NSPLIT + q],
                )
                for q in range(NSPLIT)
            ]

            class _Multi:
                def wait(self):
                    for d in descs:
                        d.wait()

            return _Multi()

        def write(ci, b):
            return pltpu.async_copy(
                rows_v.at[pl.ds(b * CB, CB)],
                out_hbm.at[pl.ds((start + ci) * CB, CB)],
                wsem[b],
            )

        g_desc = [gather(0, 0), None]
        w_desc = [None, None]
        for ci in range(MAXT):
            b = ci & 1
            g_desc[b].wait()
            if ci + 1 < MAXT:
                ob = 1 - b
                if w_desc[ob] is not None:
                    w_desc[ob].wait()
                g_desc[ob] = gather(ci + 1, ob)
            w_desc[b] = write(ci, b)
        w_desc[(MAXT - 1) & 1].wait()
        w_desc[(MAXT - 2) & 1].wait()

    return k(idx, tab128)


def kernel(edge_type, table):
    idx = edge_type.astype(jnp.int32)
    tab128 = jnp.pad(table, ((0, 0), (0, D)))
    tab128 = jnp.tile(tab128, (NREP, 1))
    out3 = _sc_embed(idx, tab128)
    return out3[:, :D]


# speedup vs baseline: 1.3417x; 1.0156x over previous
import functools

import jax
import jax.numpy as jnp
from jax import lax
from jax.experimental import pallas as pl
from jax.experimental.pallas import tpu as pltpu
from jax.experimental.pallas import tpu_sc as plsc

E = 800000
D = 64
NUM_CORES = 2
NUM_SUBCORES = 16
NW = NUM_CORES * NUM_SUBCORES
CB = 320
T = E // CB
Q, R = divmod(T, NW)
MAXT = Q + 1
GROUPS = MAXT * CB // 16
NREP = 4096
NSPLIT = 4


@jax.jit
def _sc_embed(idx, tab128):
    mesh = plsc.VectorSubcoreMesh(core_axis_name="c", subcore_axis_name="s")

    @functools.partial(
        pl.kernel,
        mesh=mesh,
        out_type=jax.ShapeDtypeStruct((E, 2 * D), jnp.float32),
        scratch_types=[
            pltpu.VMEM((MAXT * CB,), jnp.int32),
            pltpu.VMEM((2 * CB, 2 * D), jnp.float32),
            [pltpu.SemaphoreType.DMA] * (2 * NSPLIT),
            pltpu.SemaphoreType.DMA,
            pltpu.SemaphoreType.DMA,
        ],
        compiler_params=pltpu.CompilerParams(needs_layout_passes=False),
    )
    def k(idx_hbm, tab_hbm, out_hbm, idx_v, rows_v, gsems, w0, w1):
        wid = lax.axis_index("s") * NUM_CORES + lax.axis_index("c")
        start = jnp.minimum(wid * Q + jnp.minimum(wid, R), T - MAXT)

        pltpu.sync_copy(idx_hbm.at[pl.ds(start * CB, MAXT * CB)], idx_v)

        iota = lax.iota(jnp.int32, 16)

        def rewrite_body(g, carry):
            v = idx_v[pl.ds(g * 16, 16)]
            rep = jnp.bitwise_and((wid * GROUPS + g) * 16 + iota, NREP - 1)
            idx_v[pl.ds(g * 16, 16)] = jnp.bitwise_and(v, 3) + rep * 4
            return carry

        lax.fori_loop(0, GROUPS, rewrite_body, 0)

        wsem = (w0, w1)
        SP = CB // NSPLIT

        def gather(ci, b):
            descs = [
                pltpu.async_copy(
                    tab_hbm.at[idx_v.at[pl.ds(ci * CB + q * SP, SP)]],
                    rows_v.at[pl.ds(b * CB + q * SP, SP)],
                    gsems[b * NSPLIT + q],
                )
                for q in range(NSPLIT)
            ]

            class _Multi:
                def wait(self):
                    for d in descs:
                        d.wait()

            return _Multi()

        def write(ci, b):
            return pltpu.async_copy(
                rows_v.at[pl.ds(b * CB, CB)],
                out_hbm.at[pl.ds((start + ci) * CB, CB)],
                wsem[b],
            )

        g_desc = [gather(0, 0), None]
        w_desc = [None, None]
        for ci in range(MAXT):
            b = ci & 1
            g_desc[b].wait()
            if ci + 1 < MAXT:
                ob = 1 - b
                if w_desc[ob] is not None:
                    w_desc[ob].wait()
                g_desc[ob] = gather(ci + 1, ob)
            w_desc[b] = write(ci, b)
        w_desc[(MAXT - 1) & 1].wait()
        w_desc[(MAXT - 2) & 1].wait()

    return k(idx, tab128)


def kernel(edge_type, table):
    idx = edge_type.astype(jnp.int32)
    tab128 = jnp.pad(table, ((0, 0), (0, D)))
    tab128 = jnp.tile(tab128, (NREP, 1))
    out3 = _sc_embed(idx, tab128)
    return out3[:, :D]
